# Initial kernel scaffold; baseline (speedup 1.0000x reference)
import jax
import jax.numpy as jnp
from jax.experimental import pallas as pl


def kernel(position_sequence, particle_types, edge_index, params):
    raise NotImplementedError("write your pallas kernel here")


# speedup vs baseline: 2.8225x; 2.8225x over previous
import functools

import jax
import jax.numpy as jnp
from jax import lax
from jax.experimental import pallas as pl
from jax.experimental.pallas import tpu as pltpu
from jax.experimental.pallas import tpu_sc as plsc

N = 10000
E = 320000
D = 2
H = 64
K_TYPES = 9
RADIUS = 0.015
W = 128

NC = 2
NS = 16
NW = NC * NS
EPW = E // NW
CH = 80
NCHUNK = EPW // CH
NP = 10240
RPT = NP // NS

_f32 = jnp.float32


def _mesh():
    return plsc.VectorSubcoreMesh(core_axis_name="c", subcore_axis_name="s")


def _gather2(table, snd, rcv):
    @functools.partial(
        pl.kernel,
        out_type=(
            jax.ShapeDtypeStruct((E, W), _f32),
            jax.ShapeDtypeStruct((E, W), _f32),
        ),
        mesh=_mesh(),
        scratch_types=[
            pltpu.VMEM((CH,), jnp.int32),
            pltpu.VMEM((CH,), jnp.int32),
            pltpu.VMEM((CH, W), _f32),
            pltpu.VMEM((CH, W), _f32),
            pltpu.SemaphoreType.DMA,
        ],
    )
    def k(tn, s_hbm, r_hbm, gs, gr, si, ri, bs, br, sem):
        wid = lax.axis_index("s") * NC + lax.axis_index("c")
        base = wid * EPW

        def body(j, carry):
            off = base + j * CH
            pltpu.sync_copy(s_hbm.at[pl.ds(off, CH)], si)
            pltpu.sync_copy(r_hbm.at[pl.ds(off, CH)], ri)
            c0 = pltpu.async_copy(tn.at[si], bs, sem)
            c1 = pltpu.async_copy(tn.at[ri], br, sem)
            c0.wait()
            c1.wait()
            pltpu.sync_copy(bs, gs.at[pl.ds(off, CH)])
            pltpu.sync_copy(br, gr.at[pl.ds(off, CH)])
            return carry

        lax.fori_loop(0, NCHUNK, body, 0)

    return k(table, snd, rcv)


ZROWS = 128


def _segsum(edges, rcv):
    @functools.partial(
        pl.kernel,
        out_type=jax.ShapeDtypeStruct((NC, NP, W), _f32),
        mesh=_mesh(),
        scratch_types=[
            pltpu.VMEM((CH,), jnp.int32),
            pltpu.VMEM((CH, W), _f32),
            pltpu.VMEM((ZROWS, W), _f32),
            pltpu.VMEM_SHARED((NP, W), _f32),
        ],
    )
    def k(e_hbm, r_hbm, out, idx_v, ebuf, zbuf, acc):
        cid = lax.axis_index("c")
        sid = lax.axis_index("s")
        wid = sid * NC + cid

        def zrow(i, carry):
            def zcol(t, carry2):
                zbuf[i, pl.ds(t * 16, 16)] = jnp.zeros((16,), _f32)
                return carry2
            return lax.fori_loop(0, W // 16, zcol, carry)

        lax.fori_loop(0, ZROWS, zrow, 0)

        def zcopy(t, carry):
            pltpu.sync_copy(zbuf, acc.at[pl.ds(sid * RPT + t * ZROWS, ZROWS)])
            return carry

        lax.fori_loop(0, RPT // ZROWS, zcopy, 0)
        plsc.subcore_barrier()

        def body(j, carry):
            off = wid * EPW + j * CH
            pltpu.sync_copy(r_hbm.at[pl.ds(off, CH)], idx_v)
            pltpu.sync_copy(e_hbm.at[pl.ds(off, CH)], ebuf)
            pltpu.sync_copy(ebuf, acc.at[idx_v], add=True)
            return carry

        lax.fori_loop(0, NCHUNK, body, 0)
        plsc.subcore_barrier()

        pltpu.sync_copy(acc.at[pl.ds(sid * RPT, RPT)],
                        out.at[cid, pl.ds(sid * RPT, RPT)])

    return k(edges, rcv)


BN = 1000
BE = 2000


def _dot(a, b):
    return jax.lax.dot_general(a, b, (((1,), (0,)), ((), ())),
                               preferred_element_type=_f32)


def _enc_node_body(nf, oh, p16, w1a, ew, b1, w2, b2, w3, b3, out):
    h = jnp.maximum(_dot(nf[...], w1a[...]) + _dot(oh[...], ew[...]) + b1[...], 0.0)
    h = jnp.maximum(_dot(h, w2[...]) + b2[...], 0.0)
    h = _dot(h, w3[...]) + b3[...]
    z = jnp.zeros((h.shape[0], W - H - 16), _f32)
    out[...] = jnp.concatenate([h, p16[...], z], axis=1)


def _edge0_body(gs, gr, posmask, we1p, we1d, be1, we2, be2, we3, be3,
                w1e, w1s, w1r, b1, w2, b2, w3, b3, out):
    inv_r = 1.0 / RADIUS
    rel = (gs[...] - gr[...]) * (posmask[...] * inv_r)
    dist = jnp.sqrt(jnp.sum(rel * rel, axis=1, keepdims=True))
    h = jnp.maximum(_dot(rel, we1p[...]) + dist * we1d[...] + be1[...], 0.0)
    h = jnp.maximum(_dot(h, we2[...]) + be2[...], 0.0)
    e0 = _dot(h, we3[...]) + be3[...]
    h = jnp.maximum(_dot(e0, w1e[...]) + _dot(gs[...], w1s[...])
                    + _dot(gr[...], w1r[...]) + b1[...], 0.0)
    h = jnp.maximum(_dot(h, w2[...]) + b2[...], 0.0)
    out[...] = jnp.pad(e0, ((0, 0), (0, W - H))) + _dot(h, w3[...]) + b3[...]


def _edge1_body(e, gs, gr, w1e, w1s, w1r, b1, w2, b2, w3, b3, out):
    h = jnp.maximum(_dot(e[...], w1e[...]) + _dot(gs[...], w1s[...])
                    + _dot(gr[...], w1r[...]) + b1[...], 0.0)
    h = jnp.maximum(_dot(h, w2[...]) + b2[...], 0.0)
    out[...] = e[...] + _dot(h, w3[...]) + b3[...]


def _node_step_body(node, p0, p1, w1a, w1b, b1, w2, b2, w3, b3, out):
    agg = p0[...] + p1[...]
    h = jnp.maximum(_dot(node[...], w1a[...]) + _dot(agg, w1b[...]) + b1[...], 0.0)
    h = jnp.maximum(_dot(h, w2[...]) + b2[...], 0.0)
    out[...] = node[...] + _dot(h, w3[...]) + b3[...]


def _node_dec_body(node, p0, p1, w1a, w1b, b1, w2, b2, w3, b3,
                   wd1, bd1, wd2, bd2, wd3, bd3, out):
    agg = p0[...] + p1[...]
    h = jnp.maximum(_dot(node[...], w1a[...]) + _dot(agg, w1b[...]) + b1[...], 0.0)
    h = jnp.maximum(_dot(h, w2[...]) + b2[...], 0.0)
    n2 = node[...] + _dot(h, w3[...]) + b3[...]
    d = jnp.maximum(_dot(n2, wd1[...]) + bd1[...], 0.0)
    d = jnp.maximum(_dot(d, wd2[...]) + bd2[...], 0.0)
    out[...] = _dot(d, wd3[...]) + bd3[...]


def _row_spec(block, width):
    return pl.BlockSpec((block, width), lambda i: (i, 0))


def _w_spec(shape):
    return pl.BlockSpec(shape, lambda i: tuple(0 for _ in shape))


def _tc_call(body, grid, in_arrays, in_specs, out_shape, out_spec):
    return pl.pallas_call(
        body,
        grid=(grid,),
        in_specs=in_specs,
        out_specs=out_spec,
        out_shape=out_shape,
    )(*in_arrays)


def _pad_rows(w, rows):
    return jnp.zeros((rows, w.shape[1]), _f32).at[:w.shape[0]].set(w)


def _pad_cols(w, cols):
    return jnp.zeros((w.shape[0], cols), _f32).at[:, :w.shape[1]].set(w)


def kernel(position_sequence, particle_types, edge_index, params):
    num = position_sequence.shape[0]
    cp = position_sequence[:, -1, :]
    prev = position_sequence[:, -2, :]
    vel = position_sequence[:, 1:] - position_sequence[:, :-1]
    flat_nv = vel.reshape(num, -1)
    dist_b = jnp.minimum(
        jnp.concatenate([cp, 1.0 - cp], axis=-1) / RADIUS, 1.0)
    nf14 = jnp.concatenate([flat_nv, dist_b], axis=-1)
    onehot = (particle_types[:, None] == jnp.arange(K_TYPES)[None, :]).astype(_f32)
    pos16 = jnp.zeros((num, 16), _f32).at[:, :D].set(cp)

    receivers = edge_index[0]
    senders = edge_index[1]

    p = params
    (wn1, bn1), (wn2, bn2), (wn3, bn3) = p["enc_node"]
    (we1, be1), (we2, be2), (we3, be3) = p["enc_edge"]
    emb = p["type_embedding"]

    w1a = wn1[:14]
    ew = emb @ wn1[14:30]

    we1p = jnp.zeros((W, H), _f32).at[H:H + D].set(we1[:D])
    we1d = we1[D:D + 1]
    posmask = jnp.zeros((1, W), _f32).at[0, H:H + D].set(1.0)

    def r2(b):
        return b.reshape(1, -1)

    table0 = _tc_call(
        _enc_node_body, num // BN,
        [nf14, onehot, pos16, w1a, ew, r2(bn1), wn2, r2(bn2), wn3, r2(bn3)],
        [_row_spec(BN, 14), _row_spec(BN, K_TYPES), _row_spec(BN, 16),
         _w_spec((14, H)), _w_spec((K_TYPES, H)), _w_spec((1, H)),
         _w_spec((H, H)), _w_spec((1, H)), _w_spec((H, H)), _w_spec((1, H))],
        jax.ShapeDtypeStruct((num, W), _f32), _row_spec(BN, W))

    gs0, gr0 = _gather2(table0, senders, receivers)

    (pw1, pb1), (pw2, pb2), (pw3, pb3) = p["proc_edge"][0]
    edge1 = _tc_call(
        _edge0_body, E // BE,
        [gs0, gr0, posmask, we1p, we1d, r2(be1), we2, r2(be2), we3, r2(be3),
         pw1[:H], _pad_rows(pw1[H:2 * H], W), _pad_rows(pw1[2 * H:], W),
         r2(pb1), pw2, r2(pb2), _pad_cols(pw3, W), r2(_pad_cols(r2(pb3), W))],
        [_row_spec(BE, W), _row_spec(BE, W), _w_spec((1, W)), _w_spec((W, H)),
         _w_spec((1, H)), _w_spec((1, H)), _w_spec((H, H)), _w_spec((1, H)),
         _w_spec((H, H)), _w_spec((1, H)), _w_spec((H, H)), _w_spec((W, H)),
         _w_spec((W, H)), _w_spec((1, H)), _w_spec((H, H)), _w_spec((1, H)),
         _w_spec((H, W)), _w_spec((1, W))],
        jax.ShapeDtypeStruct((E, W), _f32), _row_spec(BE, W))

    part0 = _segsum(edge1, receivers)

    (nw1, nb1), (nw2, nb2), (nw3, nb3) = p["proc_node"][0]
    table1 = _tc_call(
        _node_step_body, num // BN,
        [table0, part0[0], part0[1], _pad_rows(nw1[:H], W),
         _pad_rows(nw1[H:], W), r2(nb1), nw2, r2(nb2), _pad_cols(nw3, W),
         r2(_pad_cols(r2(nb3), W))],
        [_row_spec(BN, W), _row_spec(BN, W), _row_spec(BN, W), _w_spec((W, H)),
         _w_spec((W, H)), _w_spec((1, H)), _w_spec((H, H)), _w_spec((1, H)),
         _w_spec((H, W)), _w_spec((1, W))],
        jax.ShapeDtypeStruct((num, W), _f32), _row_spec(BN, W))

    gs1, gr1 = _gather2(table1, senders, receivers)

    (qw1, qb1), (qw2, qb2), (qw3, qb3) = p["proc_edge"][1]
    edge2 = _tc_call(
        _edge1_body, E // BE,
        [edge1, gs1, gr1, _pad_rows(qw1[:H], W), _pad_rows(qw1[H:2 * H], W),
         _pad_rows(qw1[2 * H:], W), r2(qb1), qw2, r2(qb2), _pad_cols(qw3, W),
         r2(_pad_cols(r2(qb3), W))],
        [_row_spec(BE, W), _row_spec(BE, W), _row_spec(BE, W), _w_spec((W, H)),
         _w_spec((W, H)), _w_spec((W, H)), _w_spec((1, H)), _w_spec((H, H)),
         _w_spec((1, H)), _w_spec((H, W)), _w_spec((1, W))],
        jax.ShapeDtypeStruct((E, W), _f32), _row_spec(BE, W))

    part1 = _segsum(edge2, receivers)

    (mw1, mb1), (mw2, mb2), (mw3, mb3) = p["proc_node"][1]
    (dw1, db1), (dw2, db2), (dw3, db3) = p["dec"]
    accp = _tc_call(
        _node_dec_body, num // BN,
        [table1, part1[0], part1[1], _pad_rows(mw1[:H], W),
         _pad_rows(mw1[H:], W), r2(mb1), mw2, r2(mb2), _pad_cols(mw3, W),
         r2(_pad_cols(r2(mb3), W)), _pad_rows(dw1, W), r2(db1), dw2, r2(db2),
         _pad_cols(dw3, 16), r2(_pad_cols(r2(db3), 16))],
        [_row_spec(BN, W), _row_spec(BN, W), _row_spec(BN, W), _w_spec((W, H)),
         _w_spec((W, H)), _w_spec((1, H)), _w_spec((H, H)), _w_spec((1, H)),
         _w_spec((H, W)), _w_spec((1, W)), _w_spec((W, H)), _w_spec((1, H)),
         _w_spec((H, H)), _w_spec((1, H)), _w_spec((H, 16)), _w_spec((1, 16))],
        jax.ShapeDtypeStruct((num, 16), _f32), _row_spec(BN, 16))

    acceleration = accp[:, :D]
    new_velocity = (cp - prev) + acceleration
    return cp + new_velocity


# speedup vs baseline: 4.4922x; 1.5916x over previous
import functools

import jax
import jax.numpy as jnp
from jax import lax
from jax.experimental import pallas as pl
from jax.experimental.pallas import tpu as pltpu
from jax.experimental.pallas import tpu_sc as plsc

N = 10000
E = 320000
D = 2
H = 64
K_TYPES = 9
RADIUS = 0.015
W = 128

NC = 2
NS = 16
NW = NC * NS
EH = E // 2
EPW = EH // NW
CH = 40
NCHUNK = EPW // CH
PAIRS = NCHUNK // 2
TAIL = NCHUNK % 2
NP = 10240
RPT = NP // NS

_f32 = jnp.float32


def _mesh():
    return plsc.VectorSubcoreMesh(core_axis_name="c", subcore_axis_name="s")


def _gather2(table, snd, rcv, base):
    @functools.partial(
        pl.kernel,
        out_type=(
            jax.ShapeDtypeStruct((EH, W), _f32),
            jax.ShapeDtypeStruct((EH, W), _f32),
        ),
        mesh=_mesh(),
        scratch_types=[
            pltpu.VMEM((EPW,), jnp.int32),
            pltpu.VMEM((EPW,), jnp.int32),
            pltpu.VMEM((CH, W), _f32),
            pltpu.VMEM((CH, W), _f32),
            pltpu.VMEM((CH, W), _f32),
            pltpu.VMEM((CH, W), _f32),
            pltpu.SemaphoreType.DMA,
            pltpu.SemaphoreType.DMA,
            pltpu.SemaphoreType.DMA,
            pltpu.SemaphoreType.DMA,
        ],
    )
    def k(tn, s_hbm, r_hbm, gs, gr, sall, rall, bs0, br0, bs1, br1,
          gsem0, gsem1, wsem0, wsem1):
        wid = lax.axis_index("s") * NC + lax.axis_index("c")
        lb = wid * EPW
        gb = base + lb
        pltpu.sync_copy(s_hbm.at[pl.ds(gb, EPW)], sall)
        pltpu.sync_copy(r_hbm.at[pl.ds(gb, EPW)], rall)

        def sidx(j):
            return sall.at[pl.ds(j * CH, CH)]

        def ridx(j):
            return rall.at[pl.ds(j * CH, CH)]

        def issue_g(j, bs, br, sem):
            pltpu.async_copy(tn.at[sidx(j)], bs, sem)
            pltpu.async_copy(tn.at[ridx(j)], br, sem)

        def wait_g(j, bs, br, sem):
            pltpu.make_async_copy(tn.at[sidx(j)], bs, sem).wait()
            pltpu.make_async_copy(tn.at[ridx(j)], br, sem).wait()

        def issue_w(j, bs, br, sem):
            o = lb + j * CH
            pltpu.async_copy(bs, gs.at[pl.ds(o, CH)], sem)
            pltpu.async_copy(br, gr.at[pl.ds(o, CH)], sem)

        def wait_w(j, bs, br, sem):
            o = lb + j * CH
            pltpu.make_async_copy(bs, gs.at[pl.ds(o, CH)], sem).wait()
            pltpu.make_async_copy(br, gr.at[pl.ds(o, CH)], sem).wait()

        issue_g(0, bs0, br0, gsem0)

        def body(i, carry):
            j0 = 2 * i
            j1 = j0 + 1

            @pl.when(i > 0)
            def _():
                wait_w(j1 - 2, bs1, br1, wsem1)

            issue_g(j1, bs1, br1, gsem1)
            wait_g(j0, bs0, br0, gsem0)
            issue_w(j0, bs0, br0, wsem0)

            @pl.when(i < PAIRS - 1)
            def _():
                wait_w(j0, bs0, br0, wsem0)
                issue_g(j0 + 2, bs0, br0, gsem0)

            wait_g(j1, bs1, br1, gsem1)
            issue_w(j1, bs1, br1, wsem1)
            return carry

        lax.fori_loop(0, PAIRS, body, 0)
        jlast = 2 * PAIRS - 1
        wait_w(jlast - 1, bs0, br0, wsem0)
        wait_w(jlast, bs1, br1, wsem1)
        if TAIL:
            jt = NCHUNK - 1
            issue_g(jt, bs0, br0, gsem0)
            wait_g(jt, bs0, br0, gsem0)
            issue_w(jt, bs0, br0, wsem0)
            wait_w(jt, bs0, br0, wsem0)

    return k(table, snd, rcv)


ZROWS = 128


def _segsum(edges, rcv, half):
    @functools.partial(
        pl.kernel,
        out_type=jax.ShapeDtypeStruct((NC, NP, W), _f32),
        mesh=_mesh(),
        scratch_types=[
            pltpu.VMEM((CH,), jnp.int32),
            pltpu.VMEM((CH,), jnp.int32),
            pltpu.VMEM((CH, W), _f32),
            pltpu.VMEM((CH, W), _f32),
            pltpu.VMEM((ZROWS, W), _f32),
            pltpu.VMEM_SHARED((NP, W), _f32),
            pltpu.SemaphoreType.DMA,
            pltpu.SemaphoreType.DMA,
            pltpu.SemaphoreType.DMA,
            pltpu.SemaphoreType.DMA,
        ],
    )
    def k(e_hbm, r_hbm, out, ix0, ix1, eb0, eb1, zbuf, acc,
          esem0, esem1, ssem0, ssem1):
        cid = lax.axis_index("c")
        sid = lax.axis_index("s")
        wid = sid * NC + cid
        lb = wid * EPW
        gb = half * EH + lb

        def zrow(i, carry):
            def zcol(t, carry2):
                zbuf[i, pl.ds(t * 16, 16)] = jnp.zeros((16,), _f32)
                return carry2
            return lax.fori_loop(0, W // 16, zcol, carry)

        lax.fori_loop(0, ZROWS, zrow, 0)

        def zcopy(t, carry):
            pltpu.sync_copy(zbuf, acc.at[pl.ds(sid * RPT + t * ZROWS, ZROWS)])
            return carry

        lax.fori_loop(0, RPT // ZROWS, zcopy, 0)
        plsc.subcore_barrier()

        def issue_e(j, ix, eb, sem):
            pltpu.async_copy(r_hbm.at[pl.ds(gb + j * CH, CH)], ix, sem)
            pltpu.async_copy(e_hbm.at[pl.ds(lb + j * CH, CH)], eb, sem)

        def wait_e(j, ix, eb, sem):
            pltpu.make_async_copy(
                r_hbm.at[pl.ds(gb + j * CH, CH)], ix, sem).wait()
            pltpu.make_async_copy(
                e_hbm.at[pl.ds(lb + j * CH, CH)], eb, sem).wait()

        def issue_s(ix, eb, sem):
            pltpu.async_copy(eb, acc.at[ix], sem, add=True)

        def wait_s(ix, eb, sem):
            pltpu.make_async_copy(eb, acc.at[ix], sem).wait()

        issue_e(0, ix0, eb0, esem0)

        def body(i, carry):
            j0 = 2 * i
            j1 = j0 + 1

            @pl.when(i > 0)
            def _():
                wait_s(ix1, eb1, ssem1)

            issue_e(j1, ix1, eb1, esem1)
            wait_e(j0, ix0, eb0, esem0)
            issue_s(ix0, eb0, ssem0)

            @pl.when(i < PAIRS - 1)
            def _():
                wait_s(ix0, eb0, ssem0)
                issue_e(j0 + 2, ix0, eb0, esem0)

            wait_e(j1, ix1, eb1, esem1)
            issue_s(ix1, eb1, ssem1)
            return carry

        lax.fori_loop(0, PAIRS, body, 0)
        wait_s(ix0, eb0, ssem0)
        wait_s(ix1, eb1, ssem1)
        if TAIL:
            jt = NCHUNK - 1
            issue_e(jt, ix0, eb0, esem0)
            wait_e(jt, ix0, eb0, esem0)
            issue_s(ix0, eb0, ssem0)
            wait_s(ix0, eb0, ssem0)
        plsc.subcore_barrier()

        pltpu.sync_copy(acc.at[pl.ds(sid * RPT, RPT)],
                        out.at[cid, pl.ds(sid * RPT, RPT)])

    return k(edges, rcv)


BN = 1000
BE = 2000


def _dot(a, b):
    return jax.lax.dot_general(a, b, (((1,), (0,)), ((), ())),
                               preferred_element_type=_f32)


def _enc_node_body(nf, oh, p16, w1a, emb, w1b, b1, w2, b2, w3, b3, out):
    tm = _dot(oh[...], emb[...])
    h = jnp.maximum(_dot(nf[...], w1a[...]) + _dot(tm, w1b[...]) + b1[...], 0.0)
    h = jnp.maximum(_dot(h, w2[...]) + b2[...], 0.0)
    h = _dot(h, w3[...]) + b3[...]
    z = jnp.zeros((h.shape[0], W - H - 16), _f32)
    out[...] = jnp.concatenate([h, p16[...], z], axis=1)


def _edge0_body(gs, gr, posmask, we1p, we1d, be1, we2, be2, we3, be3,
                w1e, w1s, w1r, b1, w2, b2, w3, b3, out):
    rel = (gs[...] - gr[...]) * posmask[...] / RADIUS
    dist = jnp.sqrt(jnp.sum(rel * rel, axis=1, keepdims=True))
    dist_b = dist.astype(jnp.bfloat16).astype(_f32)
    wd_b = we1d[...].astype(jnp.bfloat16).astype(_f32)
    h = jnp.maximum(_dot(rel, we1p[...]) + dist_b * wd_b + be1[...], 0.0)
    h = jnp.maximum(_dot(h, we2[...]) + be2[...], 0.0)
    e0 = _dot(h, we3[...]) + be3[...]
    h = jnp.maximum(_dot(e0, w1e[...]) + _dot(gs[...], w1s[...])
                    + _dot(gr[...], w1r[...]) + b1[...], 0.0)
    h = jnp.maximum(_dot(h, w2[...]) + b2[...], 0.0)
    out[...] = jnp.pad(e0, ((0, 0), (0, W - H))) + _dot(h, w3[...]) + b3[...]


def _edge1_body(e, gs, gr, w1e, w1s, w1r, b1, w2, b2, w3, b3, out):
    h = jnp.maximum(_dot(e[...], w1e[...]) + _dot(gs[...], w1s[...])
                    + _dot(gr[...], w1r[...]) + b1[...], 0.0)
    h = jnp.maximum(_dot(h, w2[...]) + b2[...], 0.0)
    out[...] = e[...] + _dot(h, w3[...]) + b3[...]


def _node_step_body(node, p0, p1, p2, p3, w1a, w1b, b1, w2, b2, w3, b3, out):
    agg = (p0[...] + p1[...]) + (p2[...] + p3[...])
    h = jnp.maximum(_dot(node[...], w1a[...]) + _dot(agg, w1b[...]) + b1[...], 0.0)
    h = jnp.maximum(_dot(h, w2[...]) + b2[...], 0.0)
    out[...] = node[...] + _dot(h, w3[...]) + b3[...]


def _node_dec_body(node, p0, p1, p2, p3, w1a, w1b, b1, w2, b2, w3, b3,
                   wd1, bd1, wd2, bd2, wd3, bd3, out):
    agg = (p0[...] + p1[...]) + (p2[...] + p3[...])
    h = jnp.maximum(_dot(node[...], w1a[...]) + _dot(agg, w1b[...]) + b1[...], 0.0)
    h = jnp.maximum(_dot(h, w2[...]) + b2[...], 0.0)
    n2 = node[...] + _dot(h, w3[...]) + b3[...]
    d = jnp.maximum(_dot(n2, wd1[...]) + bd1[...], 0.0)
    d = jnp.maximum(_dot(d, wd2[...]) + bd2[...], 0.0)
    out[...] = _dot(d, wd3[...]) + bd3[...]


def _row_spec(block, width):
    return pl.BlockSpec((block, width), lambda i: (i, 0))


def _w_spec(shape):
    return pl.BlockSpec(shape, lambda i: tuple(0 for _ in shape))


def _tc_call(body, grid, in_arrays, in_specs, out_shape, out_spec):
    return pl.pallas_call(
        body,
        grid=(grid,),
        in_specs=in_specs,
        out_specs=out_spec,
        out_shape=out_shape,
    )(*in_arrays)


def _pad_rows(w, rows):
    return jnp.zeros((rows, w.shape[1]), _f32).at[:w.shape[0]].set(w)


def _pad_cols(w, cols):
    return jnp.zeros((w.shape[0], cols), _f32).at[:, :w.shape[1]].set(w)


def _edge_step0(gs, gr, ew_params, pe_params, masks):
    (we1p, we1d, be1, we2, be2, we3, be3) = ew_params
    (pw1, pb1), (pw2, pb2), (pw3, pb3) = pe_params
    posmask = masks

    def r2(b):
        return b.reshape(1, -1)

    return _tc_call(
        _edge0_body, EH // BE,
        [gs, gr, posmask, we1p, we1d, r2(be1), we2, r2(be2), we3, r2(be3),
         pw1[:H], _pad_rows(pw1[H:2 * H], W), _pad_rows(pw1[2 * H:], W),
         r2(pb1), pw2, r2(pb2), _pad_cols(pw3, W), r2(_pad_cols(r2(pb3), W))],
        [_row_spec(BE, W), _row_spec(BE, W), _w_spec((1, W)), _w_spec((W, H)),
         _w_spec((1, H)), _w_spec((1, H)), _w_spec((H, H)), _w_spec((1, H)),
         _w_spec((H, H)), _w_spec((1, H)), _w_spec((H, H)), _w_spec((W, H)),
         _w_spec((W, H)), _w_spec((1, H)), _w_spec((H, H)), _w_spec((1, H)),
         _w_spec((H, W)), _w_spec((1, W))],
        jax.ShapeDtypeStruct((EH, W), _f32), _row_spec(BE, W))


def _edge_step1(e, gs, gr, qe_params):
    (qw1, qb1), (qw2, qb2), (qw3, qb3) = qe_params

    def r2(b):
        return b.reshape(1, -1)

    return _tc_call(
        _edge1_body, EH // BE,
        [e, gs, gr, _pad_rows(qw1[:H], W), _pad_rows(qw1[H:2 * H], W),
         _pad_rows(qw1[2 * H:], W), r2(qb1), qw2, r2(qb2), _pad_cols(qw3, W),
         r2(_pad_cols(r2(qb3), W))],
        [_row_spec(BE, W), _row_spec(BE, W), _row_spec(BE, W), _w_spec((W, H)),
         _w_spec((W, H)), _w_spec((W, H)), _w_spec((1, H)), _w_spec((H, H)),
         _w_spec((1, H)), _w_spec((H, W)), _w_spec((1, W))],
        jax.ShapeDtypeStruct((EH, W), _f32), _row_spec(BE, W))


def kernel(position_sequence, particle_types, edge_index, params):
    num = position_sequence.shape[0]
    cp = position_sequence[:, -1, :]
    prev = position_sequence[:, -2, :]
    vel = position_sequence[:, 1:] - position_sequence[:, :-1]
    flat_nv = vel.reshape(num, -1)
    dist_b = jnp.minimum(
        jnp.concatenate([cp, 1.0 - cp], axis=-1) / RADIUS, 1.0)
    nf14 = jnp.concatenate([flat_nv, dist_b], axis=-1)
    onehot = (particle_types[:, None] == jnp.arange(K_TYPES)[None, :]).astype(_f32)
    pos16 = jnp.zeros((num, 16), _f32).at[:, :D].set(cp)

    receivers = edge_index[0]
    senders = edge_index[1]

    p = params
    (wn1, bn1), (wn2, bn2), (wn3, bn3) = p["enc_node"]
    (we1, be1), (we2, be2), (we3, be3) = p["enc_edge"]
    emb = p["type_embedding"]

    w1a = wn1[:14]
    w1b = wn1[14:30]

    we1p = jnp.zeros((W, H), _f32).at[H:H + D].set(we1[:D])
    we1d = we1[D:D + 1]
    posmask = jnp.zeros((1, W), _f32).at[0, H:H + D].set(1.0)
    ew_params = (we1p, we1d, be1, we2, be2, we3, be3)

    def r2(b):
        return b.reshape(1, -1)

    table0 = _tc_call(
        _enc_node_body, num // BN,
        [nf14, onehot, pos16, w1a, emb, w1b, r2(bn1), wn2, r2(bn2), wn3,
         r2(bn3)],
        [_row_spec(BN, 14), _row_spec(BN, K_TYPES), _row_spec(BN, 16),
         _w_spec((14, H)), _w_spec((K_TYPES, 16)), _w_spec((16, H)),
         _w_spec((1, H)),
         _w_spec((H, H)), _w_spec((1, H)), _w_spec((H, H)), _w_spec((1, H))],
        jax.ShapeDtypeStruct((num, W), _f32), _row_spec(BN, W))

    def step(table, prev_edges, pe_params, pn_params, last, dec_params=None):
        ga = _gather2(table, senders, receivers, 0)
        gb = _gather2(table, senders, receivers, EH)
        if prev_edges is None:
            ea = _edge_step0(ga[0], ga[1], ew_params, pe_params, posmask)
            eb = _edge_step0(gb[0], gb[1], ew_params, pe_params, posmask)
        else:
            ea = _edge_step1(prev_edges[0], ga[0], ga[1], pe_params)
            eb = _edge_step1(prev_edges[1], gb[0], gb[1], pe_params)
        sa = _segsum(ea, receivers, 0)
        sb = _segsum(eb, receivers, 1)
        (nw1, nb1), (nw2, nb2), (nw3, nb3) = pn_params
        common = [_pad_rows(nw1[:H], W), _pad_rows(nw1[H:], W), r2(nb1),
                  nw2, r2(nb2)]
        common_specs = [_w_spec((W, H)), _w_spec((W, H)), _w_spec((1, H)),
                        _w_spec((H, H)), _w_spec((1, H))]
        part_arrays = [sa[0], sa[1], sb[0], sb[1]]
        part_specs = [_row_spec(BN, W)] * 4
        if not last:
            new_table = _tc_call(
                _node_step_body, num // BN,
                [table] + part_arrays + common + [_pad_cols(nw3, W),
                                                 r2(_pad_cols(r2(nb3), W))],
                [_row_spec(BN, W)] + part_specs + common_specs
                + [_w_spec((H, W)), _w_spec((1, W))],
                jax.ShapeDtypeStruct((num, W), _f32), _row_spec(BN, W))
            return new_table, (ea, eb)
        (dw1, db1), (dw2, db2), (dw3, db3) = dec_params
        accp = _tc_call(
            _node_dec_body, num // BN,
            [table] + part_arrays + common + [
                _pad_cols(nw3, W), r2(_pad_cols(r2(nb3), W)),
                _pad_rows(dw1, W), r2(db1), dw2, r2(db2),
                _pad_cols(dw3, 16), r2(_pad_cols(r2(db3), 16))],
            [_row_spec(BN, W)] + part_specs + common_specs + [
                _w_spec((H, W)), _w_spec((1, W)), _w_spec((W, H)),
                _w_spec((1, H)), _w_spec((H, H)), _w_spec((1, H)),
                _w_spec((H, 16)), _w_spec((1, 16))],
            jax.ShapeDtypeStruct((num, 16), _f32), _row_spec(BN, 16))
        return accp, None

    table1, edges1 = step(table0, None, p["proc_edge"][0], p["proc_node"][0],
                          last=False)
    accp, _ = step(table1, edges1, p["proc_edge"][1], p["proc_node"][1],
                   last=True, dec_params=p["dec"])

    acceleration = accp[:, :D]
    new_velocity = (cp - prev) + acceleration
    return cp + new_velocity


# speedup vs baseline: 4.6605x; 1.0375x over previous
import functools

import jax
import jax.numpy as jnp
from jax import lax
from jax.experimental import pallas as pl
from jax.experimental.pallas import tpu as pltpu
from jax.experimental.pallas import tpu_sc as plsc

N = 10000
E = 320000
D = 2
H = 64
K_TYPES = 9
RADIUS = 0.015
W = 128

NC = 2
NS = 16
NW = NC * NS
EH = E // 2
EPW = EH // NW
CH = 128
NF = EPW // CH
REM = EPW % CH
PAIRS = NF // 2
LEFT = NF % 2
NP = 10240
RPT = NP // NS

_f32 = jnp.float32


def _mesh():
    return plsc.VectorSubcoreMesh(core_axis_name="c", subcore_axis_name="s")


def _gather2(table, snd, rcv, base):
    @functools.partial(
        pl.kernel,
        out_type=(
            jax.ShapeDtypeStruct((EH, W), _f32),
            jax.ShapeDtypeStruct((EH, W), _f32),
        ),
        mesh=_mesh(),
        scratch_types=[
            pltpu.VMEM((EPW,), jnp.int32),
            pltpu.VMEM((EPW,), jnp.int32),
            pltpu.VMEM((CH, W), _f32),
            pltpu.VMEM((CH, W), _f32),
            pltpu.VMEM((CH, W), _f32),
            pltpu.VMEM((CH, W), _f32),
            pltpu.SemaphoreType.DMA,
            pltpu.SemaphoreType.DMA,
            pltpu.SemaphoreType.DMA,
            pltpu.SemaphoreType.DMA,
        ],
    )
    def k(tn, s_hbm, r_hbm, gs, gr, sall, rall, bs0, br0, bs1, br1,
          gsem0, gsem1, wsem0, wsem1):
        wid = lax.axis_index("s") * NC + lax.axis_index("c")
        lb = wid * EPW
        gb = base + lb
        pltpu.sync_copy(s_hbm.at[pl.ds(gb, EPW)], sall)
        pltpu.sync_copy(r_hbm.at[pl.ds(gb, EPW)], rall)

        def sidx(j, n):
            return sall.at[pl.ds(j * CH, n)]

        def ridx(j, n):
            return rall.at[pl.ds(j * CH, n)]

        def sub(buf, n):
            return buf if n == CH else buf.at[pl.ds(0, n)]

        def issue_g(j, n, bs, br, sem):
            pltpu.async_copy(tn.at[sidx(j, n)], sub(bs, n), sem)
            pltpu.async_copy(tn.at[ridx(j, n)], sub(br, n), sem)

        def wait_g(j, n, bs, br, sem):
            pltpu.make_async_copy(tn.at[sidx(j, n)], sub(bs, n), sem).wait()
            pltpu.make_async_copy(tn.at[ridx(j, n)], sub(br, n), sem).wait()

        def issue_w(j, n, bs, br, sem):
            o = lb + j * CH
            pltpu.async_copy(sub(bs, n), gs.at[pl.ds(o, n)], sem)
            pltpu.async_copy(sub(br, n), gr.at[pl.ds(o, n)], sem)

        def wait_w(j, n, bs, br, sem):
            o = lb + j * CH
            pltpu.make_async_copy(sub(bs, n), gs.at[pl.ds(o, n)], sem).wait()
            pltpu.make_async_copy(sub(br, n), gr.at[pl.ds(o, n)], sem).wait()

        bufs = ((bs0, br0, gsem0, wsem0), (bs1, br1, gsem1, wsem1))

        issue_g(0, CH, bs0, br0, gsem0)

        def body(i, carry):
            j0 = 2 * i
            j1 = j0 + 1

            @pl.when(i > 0)
            def _():
                wait_w(j1 - 2, CH, bs1, br1, wsem1)

            issue_g(j1, CH, bs1, br1, gsem1)
            wait_g(j0, CH, bs0, br0, gsem0)
            issue_w(j0, CH, bs0, br0, wsem0)

            @pl.when(i < PAIRS - 1)
            def _():
                wait_w(j0, CH, bs0, br0, wsem0)
                issue_g(j0 + 2, CH, bs0, br0, gsem0)

            wait_g(j1, CH, bs1, br1, gsem1)
            issue_w(j1, CH, bs1, br1, wsem1)
            return carry

        lax.fori_loop(0, PAIRS, body, 0)
        pend = {0: (2 * PAIRS - 2, CH), 1: (2 * PAIRS - 1, CH)}
        extra = ([(2 * PAIRS, CH)] if LEFT else []) + ([(NF, REM)] if REM else [])
        b = 0
        for (j, n) in extra:
            bs, br, gsm, wsm = bufs[b]
            wait_w(pend[b][0], pend[b][1], bs, br, wsm)
            issue_g(j, n, bs, br, gsm)
            b ^= 1
        b = 0
        for (j, n) in extra:
            bs, br, gsm, wsm = bufs[b]
            wait_g(j, n, bs, br, gsm)
            issue_w(j, n, bs, br, wsm)
            pend[b] = (j, n)
            b ^= 1
        for b in (0, 1):
            bs, br, gsm, wsm = bufs[b]
            wait_w(pend[b][0], pend[b][1], bs, br, wsm)

    return k(table, snd, rcv)


ZROWS = 64


def _segsum(edges, rcv, half):
    @functools.partial(
        pl.kernel,
        out_type=jax.ShapeDtypeStruct((NC, NP, W), _f32),
        mesh=_mesh(),
        scratch_types=[
            pltpu.VMEM((CH,), jnp.int32),
            pltpu.VMEM((CH,), jnp.int32),
            pltpu.VMEM((CH, W), _f32),
            pltpu.VMEM((CH, W), _f32),
            pltpu.VMEM((ZROWS, W), _f32),
            pltpu.VMEM_SHARED((NP, W), _f32),
            pltpu.SemaphoreType.DMA,
            pltpu.SemaphoreType.DMA,
            pltpu.SemaphoreType.DMA,
            pltpu.SemaphoreType.DMA,
        ],
    )
    def k(e_hbm, r_hbm, out, ix0, ix1, eb0, eb1, zbuf, acc,
          esem0, esem1, ssem0, ssem1):
        cid = lax.axis_index("c")
        sid = lax.axis_index("s")
        wid = sid * NC + cid
        lb = wid * EPW
        gb = half * EH + lb

        def zrow(i, carry):
            def zcol(t, carry2):
                zbuf[i, pl.ds(t * 16, 16)] = jnp.zeros((16,), _f32)
                return carry2
            return lax.fori_loop(0, W // 16, zcol, carry)

        lax.fori_loop(0, ZROWS, zrow, 0)

        def zcopy(t, carry):
            pltpu.sync_copy(zbuf, acc.at[pl.ds(sid * RPT + t * ZROWS, ZROWS)])
            return carry

        lax.fori_loop(0, RPT // ZROWS, zcopy, 0)
        plsc.subcore_barrier()

        def sub(buf, n):
            return buf if n == CH else buf.at[pl.ds(0, n)]

        def issue_e(j, n, ix, eb, sem):
            pltpu.async_copy(r_hbm.at[pl.ds(gb + j * CH, n)], sub(ix, n), sem)
            pltpu.async_copy(e_hbm.at[pl.ds(lb + j * CH, n)], sub(eb, n), sem)

        def wait_e(j, n, ix, eb, sem):
            pltpu.make_async_copy(
                r_hbm.at[pl.ds(gb + j * CH, n)], sub(ix, n), sem).wait()
            pltpu.make_async_copy(
                e_hbm.at[pl.ds(lb + j * CH, n)], sub(eb, n), sem).wait()

        def issue_s(n, ix, eb, sem):
            pltpu.async_copy(sub(eb, n), acc.at[sub(ix, n)], sem, add=True)

        def wait_s(n, ix, eb, sem):
            pltpu.make_async_copy(sub(eb, n), acc.at[sub(ix, n)], sem).wait()

        bufs = ((ix0, eb0, esem0, ssem0), (ix1, eb1, esem1, ssem1))

        issue_e(0, CH, ix0, eb0, esem0)

        def body(i, carry):
            j0 = 2 * i
            j1 = j0 + 1

            @pl.when(i > 0)
            def _():
                wait_s(CH, ix1, eb1, ssem1)

            issue_e(j1, CH, ix1, eb1, esem1)
            wait_e(j0, CH, ix0, eb0, esem0)
            issue_s(CH, ix0, eb0, ssem0)

            @pl.when(i < PAIRS - 1)
            def _():
                wait_s(CH, ix0, eb0, ssem0)
                issue_e(j0 + 2, CH, ix0, eb0, esem0)

            wait_e(j1, CH, ix1, eb1, esem1)
            issue_s(CH, ix1, eb1, ssem1)
            return carry

        lax.fori_loop(0, PAIRS, body, 0)
        pend = {0: CH, 1: CH}
        extra = ([(2 * PAIRS, CH)] if LEFT else []) + ([(NF, REM)] if REM else [])
        b = 0
        for (j, n) in extra:
            ix, eb, esm, ssm = bufs[b]
            wait_s(pend[b], ix, eb, ssm)
            issue_e(j, n, ix, eb, esm)
            b ^= 1
        b = 0
        for (j, n) in extra:
            ix, eb, esm, ssm = bufs[b]
            wait_e(j, n, ix, eb, esm)
            issue_s(n, ix, eb, ssm)
            pend[b] = n
            b ^= 1
        for b in (0, 1):
            ix, eb, esm, ssm = bufs[b]
            wait_s(pend[b], ix, eb, ssm)
        plsc.subcore_barrier()

        pltpu.sync_copy(acc.at[pl.ds(sid * RPT, RPT)],
                        out.at[cid, pl.ds(sid * RPT, RPT)])

    return k(edges, rcv)


BN = 1000
BE = 2000


def _dot(a, b):
    return jax.lax.dot_general(a, b, (((1,), (0,)), ((), ())),
                               preferred_element_type=_f32)


def _enc_node_body(nf, oh, p16, w1a, emb, w1b, b1, w2, b2, w3, b3, out):
    tm = _dot(oh[...], emb[...])
    h = jnp.maximum(_dot(nf[...], w1a[...]) + _dot(tm, w1b[...]) + b1[...], 0.0)
    h = jnp.maximum(_dot(h, w2[...]) + b2[...], 0.0)
    h = _dot(h, w3[...]) + b3[...]
    z = jnp.zeros((h.shape[0], W - H - 16), _f32)
    out[...] = jnp.concatenate([h, p16[...], z], axis=1)


def _edge0_body(gs, gr, posmask, we1p, we1d, be1, we2, be2, we3, be3,
                w1e, w1s, w1r, b1, w2, b2, w3, b3, out):
    rel = (gs[...] - gr[...]) * posmask[...] / RADIUS
    dist = jnp.sqrt(jnp.sum(rel * rel, axis=1, keepdims=True))
    dist_b = dist.astype(jnp.bfloat16).astype(_f32)
    wd_b = we1d[...].astype(jnp.bfloat16).astype(_f32)
    h = jnp.maximum(_dot(rel, we1p[...]) + dist_b * wd_b + be1[...], 0.0)
    h = jnp.maximum(_dot(h, we2[...]) + be2[...], 0.0)
    e0 = _dot(h, we3[...]) + be3[...]
    h = jnp.maximum(_dot(e0, w1e[...]) + _dot(gs[...], w1s[...])
                    + _dot(gr[...], w1r[...]) + b1[...], 0.0)
    h = jnp.maximum(_dot(h, w2[...]) + b2[...], 0.0)
    out[...] = jnp.pad(e0, ((0, 0), (0, W - H))) + _dot(h, w3[...]) + b3[...]


def _edge1_body(e, gs, gr, w1e, w1s, w1r, b1, w2, b2, w3, b3, out):
    h = jnp.maximum(_dot(e[...], w1e[...]) + _dot(gs[...], w1s[...])
                    + _dot(gr[...], w1r[...]) + b1[...], 0.0)
    h = jnp.maximum(_dot(h, w2[...]) + b2[...], 0.0)
    out[...] = e[...] + _dot(h, w3[...]) + b3[...]


def _node_step_body(node, p0, p1, p2, p3, w1a, w1b, b1, w2, b2, w3, b3, out):
    agg = (p0[...] + p1[...]) + (p2[...] + p3[...])
    h = jnp.maximum(_dot(node[...], w1a[...]) + _dot(agg, w1b[...]) + b1[...], 0.0)
    h = jnp.maximum(_dot(h, w2[...]) + b2[...], 0.0)
    out[...] = node[...] + _dot(h, w3[...]) + b3[...]


def _node_dec_body(node, p0, p1, p2, p3, w1a, w1b, b1, w2, b2, w3, b3,
                   wd1, bd1, wd2, bd2, wd3, bd3, out):
    agg = (p0[...] + p1[...]) + (p2[...] + p3[...])
    h = jnp.maximum(_dot(node[...], w1a[...]) + _dot(agg, w1b[...]) + b1[...], 0.0)
    h = jnp.maximum(_dot(h, w2[...]) + b2[...], 0.0)
    n2 = node[...] + _dot(h, w3[...]) + b3[...]
    d = jnp.maximum(_dot(n2, wd1[...]) + bd1[...], 0.0)
    d = jnp.maximum(_dot(d, wd2[...]) + bd2[...], 0.0)
    out[...] = _dot(d, wd3[...]) + bd3[...]


def _row_spec(block, width):
    return pl.BlockSpec((block, width), lambda i: (i, 0))


def _w_spec(shape):
    return pl.BlockSpec(shape, lambda i: tuple(0 for _ in shape))


def _tc_call(body, grid, in_arrays, in_specs, out_shape, out_spec):
    return pl.pallas_call(
        body,
        grid=(grid,),
        in_specs=in_specs,
        out_specs=out_spec,
        out_shape=out_shape,
    )(*in_arrays)


def _pad_rows(w, rows):
    return jnp.zeros((rows, w.shape[1]), _f32).at[:w.shape[0]].set(w)


def _pad_cols(w, cols):
    return jnp.zeros((w.shape[0], cols), _f32).at[:, :w.shape[1]].set(w)


def _edge_step0(gs, gr, ew_params, pe_params, masks):
    (we1p, we1d, be1, we2, be2, we3, be3) = ew_params
    (pw1, pb1), (pw2, pb2), (pw3, pb3) = pe_params
    posmask = masks

    def r2(b):
        return b.reshape(1, -1)

    return _tc_call(
        _edge0_body, EH // BE,
        [gs, gr, posmask, we1p, we1d, r2(be1), we2, r2(be2), we3, r2(be3),
         pw1[:H], _pad_rows(pw1[H:2 * H], W), _pad_rows(pw1[2 * H:], W),
         r2(pb1), pw2, r2(pb2), _pad_cols(pw3, W), r2(_pad_cols(r2(pb3), W))],
        [_row_spec(BE, W), _row_spec(BE, W), _w_spec((1, W)), _w_spec((W, H)),
         _w_spec((1, H)), _w_spec((1, H)), _w_spec((H, H)), _w_spec((1, H)),
         _w_spec((H, H)), _w_spec((1, H)), _w_spec((H, H)), _w_spec((W, H)),
         _w_spec((W, H)), _w_spec((1, H)), _w_spec((H, H)), _w_spec((1, H)),
         _w_spec((H, W)), _w_spec((1, W))],
        jax.ShapeDtypeStruct((EH, W), _f32), _row_spec(BE, W))


def _edge_step1(e, gs, gr, qe_params):
    (qw1, qb1), (qw2, qb2), (qw3, qb3) = qe_params

    def r2(b):
        return b.reshape(1, -1)

    return _tc_call(
        _edge1_body, EH // BE,
        [e, gs, gr, _pad_rows(qw1[:H], W), _pad_rows(qw1[H:2 * H], W),
         _pad_rows(qw1[2 * H:], W), r2(qb1), qw2, r2(qb2), _pad_cols(qw3, W),
         r2(_pad_cols(r2(qb3), W))],
        [_row_spec(BE, W), _row_spec(BE, W), _row_spec(BE, W), _w_spec((W, H)),
         _w_spec((W, H)), _w_spec((W, H)), _w_spec((1, H)), _w_spec((H, H)),
         _w_spec((1, H)), _w_spec((H, W)), _w_spec((1, W))],
        jax.ShapeDtypeStruct((EH, W), _f32), _row_spec(BE, W))


def kernel(position_sequence, particle_types, edge_index, params):
    num = position_sequence.shape[0]
    cp = position_sequence[:, -1, :]
    prev = position_sequence[:, -2, :]
    vel = position_sequence[:, 1:] - position_sequence[:, :-1]
    flat_nv = vel.reshape(num, -1)
    dist_b = jnp.minimum(
        jnp.concatenate([cp, 1.0 - cp], axis=-1) / RADIUS, 1.0)
    nf14 = jnp.concatenate([flat_nv, dist_b], axis=-1)
    onehot = (particle_types[:, None] == jnp.arange(K_TYPES)[None, :]).astype(_f32)
    pos16 = jnp.zeros((num, 16), _f32).at[:, :D].set(cp)

    receivers = edge_index[0]
    senders = edge_index[1]

    p = params
    (wn1, bn1), (wn2, bn2), (wn3, bn3) = p["enc_node"]
    (we1, be1), (we2, be2), (we3, be3) = p["enc_edge"]
    emb = p["type_embedding"]

    w1a = wn1[:14]
    w1b = wn1[14:30]

    we1p = jnp.zeros((W, H), _f32).at[H:H + D].set(we1[:D])
    we1d = we1[D:D + 1]
    posmask = jnp.zeros((1, W), _f32).at[0, H:H + D].set(1.0)
    ew_params = (we1p, we1d, be1, we2, be2, we3, be3)

    def r2(b):
        return b.reshape(1, -1)

    table0 = _tc_call(
        _enc_node_body, num // BN,
        [nf14, onehot, pos16, w1a, emb, w1b, r2(bn1), wn2, r2(bn2), wn3,
         r2(bn3)],
        [_row_spec(BN, 14), _row_spec(BN, K_TYPES), _row_spec(BN, 16),
         _w_spec((14, H)), _w_spec((K_TYPES, 16)), _w_spec((16, H)),
         _w_spec((1, H)),
         _w_spec((H, H)), _w_spec((1, H)), _w_spec((H, H)), _w_spec((1, H))],
        jax.ShapeDtypeStruct((num, W), _f32), _row_spec(BN, W))

    def step(table, prev_edges, pe_params, pn_params, last, dec_params=None):
        ga = _gather2(table, senders, receivers, 0)
        gb = _gather2(table, senders, receivers, EH)
        if prev_edges is None:
            ea = _edge_step0(ga[0], ga[1], ew_params, pe_params, posmask)
            eb = _edge_step0(gb[0], gb[1], ew_params, pe_params, posmask)
        else:
            ea = _edge_step1(prev_edges[0], ga[0], ga[1], pe_params)
            eb = _edge_step1(prev_edges[1], gb[0], gb[1], pe_params)
        sa = _segsum(ea, receivers, 0)
        sb = _segsum(eb, receivers, 1)
        (nw1, nb1), (nw2, nb2), (nw3, nb3) = pn_params
        common = [_pad_rows(nw1[:H], W), _pad_rows(nw1[H:], W), r2(nb1),
                  nw2, r2(nb2)]
        common_specs = [_w_spec((W, H)), _w_spec((W, H)), _w_spec((1, H)),
                        _w_spec((H, H)), _w_spec((1, H))]
        part_arrays = [sa[0], sa[1], sb[0], sb[1]]
        part_specs = [_row_spec(BN, W)] * 4
        if not last:
            new_table = _tc_call(
                _node_step_body, num // BN,
                [table] + part_arrays + common + [_pad_cols(nw3, W),
                                                 r2(_pad_cols(r2(nb3), W))],
                [_row_spec(BN, W)] + part_specs + common_specs
                + [_w_spec((H, W)), _w_spec((1, W))],
                jax.ShapeDtypeStruct((num, W), _f32), _row_spec(BN, W))
            return new_table, (ea, eb)
        (dw1, db1), (dw2, db2), (dw3, db3) = dec_params
        accp = _tc_call(
            _node_dec_body, num // BN,
            [table] + part_arrays + common + [
                _pad_cols(nw3, W), r2(_pad_cols(r2(nb3), W)),
                _pad_rows(dw1, W), r2(db1), dw2, r2(db2),
                _pad_cols(dw3, 16), r2(_pad_cols(r2(db3), 16))],
            [_row_spec(BN, W)] + part_specs + common_specs + [
                _w_spec((H, W)), _w_spec((1, W)), _w_spec((W, H)),
                _w_spec((1, H)), _w_spec((H, H)), _w_spec((1, H)),
                _w_spec((H, 16)), _w_spec((1, 16))],
            jax.ShapeDtypeStruct((num, 16), _f32), _row_spec(BN, 16))
        return accp, None

    table1, edges1 = step(table0, None, p["proc_edge"][0], p["proc_node"][0],
                          last=False)
    accp, _ = step(table1, edges1, p["proc_edge"][1], p["proc_node"][1],
                   last=True, dec_params=p["dec"])

    acceleration = accp[:, :D]
    new_velocity = (cp - prev) + acceleration
    return cp + new_velocity
